# baseline (speedup 1.0000x reference)
import functools

import jax
import jax.numpy as jnp
from jax import lax
from jax.experimental import pallas as pl
from jax.experimental.pallas import tpu as pltpu
from jax.experimental.pallas import tpu_sc as plsc


def kernel(node_j, node_k, phi, prob_tensor):
    num_nodes, embed = phi.shape
    nchunk = embed // 16
    fdtype = phi.dtype
    mesh = plsc.VectorSubcoreMesh(
        core_axis_name="c", subcore_axis_name="s", num_cores=1,
        num_subcores=1)

    @functools.partial(
        pl.kernel,
        out_type=jax.ShapeDtypeStruct((1,), fdtype),
        mesh=mesh,
        compiler_params=pltpu.CompilerParams(needs_layout_passes=False),
        scratch_types=[
            pltpu.VMEM((1,), jnp.int32),
            pltpu.VMEM((16,), jnp.int32),
            pltpu.VMEM((1, embed), fdtype),
            pltpu.VMEM((16, embed), fdtype),
            pltpu.VMEM((256,), fdtype),
            pltpu.VMEM((16,), fdtype),
            pltpu.SemaphoreType.DMA,
            pltpu.SemaphoreType.DMA,
        ],
    )
    def run(node_j_hbm, node_k_hbm, phi_hbm, prob_hbm, out_hbm,
            jv_v, kv_v, h_v, rows_v, part_v, f_v, sem, sem2):
        if True:
            j_cp = pltpu.async_copy(node_j_hbm, jv_v, sem)
            k_cp = pltpu.async_copy(node_k_hbm, kv_v.at[pl.ds(0, 1)], sem2)
            k_cp.wait()

            w = kv_v[...][0] + num_nodes
            w_vec = jnp.broadcast_to(w, (16,))
            iota = lax.iota(jnp.int32, 16)
            idx_vec = lax.shift_right_logical(w_vec, iota + 1)
            rows_cp = pltpu.async_copy(prob_hbm.at[idx_vec], rows_v, sem2)
            j_cp.wait()
            h_cp = pltpu.async_copy(phi_hbm.at[jv_v], h_v, sem)

            bits = lax.shift_right_logical(w_vec, iota) & 1
            sign_f = (1 - 2 * bits).astype(fdtype)
            active = idx_vec >= 2

            h_cp.wait()
            rows_cp.wait()

            hc = [h_v[0, pl.ds(16 * c, 16)] for c in range(nchunk)]

            @functools.partial(plsc.parallel_loop, 0, 16, unroll=4)
            def _row_body(r):
                acc = rows_v[r, pl.ds(0, 16)] * hc[0]
                for c in range(1, nchunk):
                    acc = acc + rows_v[r, pl.ds(16 * c, 16)] * hc[c]
                part_v[pl.ds(pl.multiple_of(16 * r, 16), 16)] = acc

            row_base = iota * 16

            def col_body(k, acc):
                return acc + plsc.load_gather(part_v, [row_base + k])

            dots = lax.fori_loop(
                1, 16, col_body, plsc.load_gather(part_v, [row_base]))

            x = sign_f * dots
            f = 1.0 / (1.0 + jnp.exp(-x))
            f = jnp.where(active, f, jnp.ones((16,), fdtype))

            f_v[...] = f
            v = f
            for step in (8, 4, 2, 1):
                v = v * plsc.load_gather(f_v, [iota ^ step])
                f_v[...] = v
            pltpu.sync_copy(f_v.at[pl.ds(0, 1)], out_hbm)

    return run(node_j, node_k, phi, prob_tensor)


# speedup vs baseline: 1.0319x; 1.0319x over previous
import functools

import jax
import jax.numpy as jnp
from jax import lax
from jax.experimental import pallas as pl
from jax.experimental.pallas import tpu as pltpu
from jax.experimental.pallas import tpu_sc as plsc


def kernel(node_j, node_k, phi, prob_tensor):
    num_nodes, embed = phi.shape
    nchunk = embed // 16
    fdtype = phi.dtype
    mesh = plsc.VectorSubcoreMesh(
        core_axis_name="c", subcore_axis_name="s", num_cores=1,
        num_subcores=1)

    @functools.partial(
        pl.kernel,
        out_type=jax.ShapeDtypeStruct((1,), fdtype),
        mesh=mesh,
        compiler_params=pltpu.CompilerParams(needs_layout_passes=False),
        scratch_types=[
            pltpu.VMEM((1,), jnp.int32),
            pltpu.VMEM((16,), jnp.int32),
            pltpu.VMEM((1, embed), fdtype),
            pltpu.VMEM((16, embed), fdtype),
            pltpu.VMEM((256,), fdtype),
            pltpu.VMEM((16,), fdtype),
            pltpu.SemaphoreType.DMA,
            pltpu.SemaphoreType.DMA,
        ],
    )
    def run(node_j_hbm, node_k_hbm, phi_hbm, prob_hbm, out_hbm,
            jv_v, kv_v, h_v, rows_v, part_v, f_v, sem, sem2):
        if True:
            j_cp = pltpu.async_copy(node_j_hbm, jv_v, sem)
            k_cp = pltpu.async_copy(node_k_hbm, kv_v.at[pl.ds(0, 1)], sem2)
            k_cp.wait()

            w = kv_v[...][0] + num_nodes
            w_vec = jnp.broadcast_to(w, (16,))
            iota = lax.iota(jnp.int32, 16)
            idx_vec = lax.shift_right_logical(w_vec, iota + 1)
            rows_cp = pltpu.async_copy(prob_hbm.at[idx_vec], rows_v, sem2)
            j_cp.wait()
            h_cp = pltpu.async_copy(phi_hbm.at[jv_v], h_v, sem)

            bits = lax.shift_right_logical(w_vec, iota) & 1
            sign_f = (1 - 2 * bits).astype(fdtype)
            active = idx_vec >= 2

            h_cp.wait()
            rows_cp.wait()

            hc = [h_v[0, pl.ds(16 * c, 16)] for c in range(nchunk)]

            @functools.partial(plsc.parallel_loop, 0, 16, unroll=4)
            def _row_body(r):
                acc = rows_v[r, pl.ds(0, 16)] * hc[0]
                for c in range(1, nchunk):
                    acc = acc + rows_v[r, pl.ds(16 * c, 16)] * hc[c]
                part_v[pl.ds(pl.multiple_of(16 * r, 16), 16)] = acc

            row_base = iota * 16
            cols = [plsc.load_gather(part_v, [row_base + k])
                    for k in range(16)]
            while len(cols) > 1:
                cols = [cols[i] + cols[i + 1]
                        for i in range(0, len(cols), 2)]
            dots = cols[0]

            x = sign_f * dots
            f = 1.0 / (1.0 + jnp.exp(-x))
            f = jnp.where(active, f, jnp.ones((16,), fdtype))

            f_v[...] = f
            v = f
            for step in (8, 4, 2, 1):
                v = v * plsc.load_gather(f_v, [iota ^ step])
                f_v[...] = v
            pltpu.sync_copy(f_v.at[pl.ds(0, 1)], out_hbm)

    return run(node_j, node_k, phi, prob_tensor)
